# baseline (speedup 1.0000x reference)
import functools

import jax
import jax.numpy as jnp
from jax.experimental import pallas as pl
from jax.experimental.pallas import tpu as pltpu

N = 4096
D_IN = 256
D_OUT = 256

BM1 = 512
BM = 1024
BK = 2048


def _support_body(att_ref, x_ref, w_ref, s_ref, *, num_rel):
    r = pl.program_id(0)
    m = att_ref[0]
    for j in range(1, num_rel):
        m = jnp.maximum(m, att_ref[j])
    denom = jnp.exp(att_ref[0] - m)
    for j in range(1, num_rel):
        denom = denom + jnp.exp(att_ref[j] - m)
    att_r = jnp.exp(att_ref[r] - m) / denom
    s_ref[0] = jnp.dot(x_ref[...], w_ref[0],
                       preferred_element_type=jnp.float32) * att_r


def _aggregate_body(a_ref, s_ref, b_ref, o_ref, *, num_rel, num_k):
    r = pl.program_id(1)
    k = pl.program_id(2)

    @pl.when((r == 0) & (k == 0))
    def _init():
        o_ref[...] = jnp.zeros_like(o_ref)

    o_ref[...] += jnp.dot(a_ref[0], s_ref[0],
                          preferred_element_type=jnp.float32)

    @pl.when((r == num_rel - 1) & (k == num_k - 1))
    def _finish():
        o_ref[...] += b_ref[...]


def kernel(input, adjs, adj_weight, attention, bias):
    num_rel, n, _ = adjs.shape
    d_in = input.shape[1]
    d_out = adj_weight.shape[2]

    support = pl.pallas_call(
        functools.partial(_support_body, num_rel=num_rel),
        grid=(num_rel, n // BM1),
        in_specs=[
            pl.BlockSpec(memory_space=pltpu.SMEM),
            pl.BlockSpec((BM1, d_in), lambda r, i: (i, 0)),
            pl.BlockSpec((1, d_in, d_out), lambda r, i: (r, 0, 0)),
        ],
        out_specs=pl.BlockSpec((1, BM1, d_out), lambda r, i: (r, i, 0)),
        out_shape=jax.ShapeDtypeStruct((num_rel, n, d_out), jnp.float32),
        compiler_params=pltpu.CompilerParams(
            dimension_semantics=("arbitrary", "parallel"),
        ),
    )(attention, input, adj_weight)

    num_k = n // BK
    bias2 = bias.reshape(1, d_out)
    out = pl.pallas_call(
        functools.partial(_aggregate_body, num_rel=num_rel, num_k=num_k),
        grid=(n // BM, num_rel, num_k),
        in_specs=[
            pl.BlockSpec((1, BM, BK), lambda i, r, k: (r, i, k)),
            pl.BlockSpec((1, BK, d_out), lambda i, r, k: (r, k, 0)),
            pl.BlockSpec((1, d_out), lambda i, r, k: (0, 0)),
        ],
        out_specs=pl.BlockSpec((BM, d_out), lambda i, r, k: (i, 0)),
        out_shape=jax.ShapeDtypeStruct((n, d_out), jnp.float32),
        compiler_params=pltpu.CompilerParams(
            dimension_semantics=("parallel", "arbitrary", "arbitrary"),
        ),
    )(adjs, support, bias2)
    return out


# speedup vs baseline: 1.3913x; 1.3913x over previous
import functools

import jax
import jax.numpy as jnp
from jax.experimental import pallas as pl
from jax.experimental.pallas import tpu as pltpu

BM = 1024
BK = 2048


def _fused_body(att_ref, x_ref, w_ref, a_ref, b_ref, o_ref, s_ref,
                *, num_rel, num_k):
    r = pl.program_id(1)
    k = pl.program_id(2)

    @pl.when((r == 0) & (k == 0))
    def _compute_support():
        m = att_ref[0]
        for j in range(1, num_rel):
            m = jnp.maximum(m, att_ref[j])
        denom = jnp.exp(att_ref[0] - m)
        for j in range(1, num_rel):
            denom = denom + jnp.exp(att_ref[j] - m)
        x = x_ref[...]
        for j in range(num_rel):
            att_j = jnp.exp(att_ref[j] - m) / denom
            s_ref[j] = jnp.dot(x, w_ref[j],
                               preferred_element_type=jnp.float32) * att_j
        o_ref[...] = jnp.broadcast_to(b_ref[...], o_ref.shape)

    o_ref[...] += jnp.dot(a_ref[0], s_ref[r, pl.ds(k * BK, BK), :],
                          preferred_element_type=jnp.float32)


def kernel(input, adjs, adj_weight, attention, bias):
    num_rel, n, _ = adjs.shape
    d_in = input.shape[1]
    d_out = adj_weight.shape[2]
    num_k = n // BK

    out = pl.pallas_call(
        functools.partial(_fused_body, num_rel=num_rel, num_k=num_k),
        grid=(n // BM, num_rel, num_k),
        in_specs=[
            pl.BlockSpec(memory_space=pltpu.SMEM),
            pl.BlockSpec((n, d_in), lambda i, r, k: (0, 0)),
            pl.BlockSpec((num_rel, d_in, d_out), lambda i, r, k: (0, 0, 0)),
            pl.BlockSpec((1, BM, BK), lambda i, r, k: (r, i, k)),
            pl.BlockSpec((1, d_out), lambda i, r, k: (0, 0)),
        ],
        out_specs=pl.BlockSpec((BM, d_out), lambda i, r, k: (i, 0)),
        out_shape=jax.ShapeDtypeStruct((n, d_out), jnp.float32),
        scratch_shapes=[pltpu.VMEM((num_rel, n, d_out), jnp.float32)],
        compiler_params=pltpu.CompilerParams(
            dimension_semantics=("parallel", "arbitrary", "arbitrary"),
        ),
    )(attention, input, adj_weight, adjs, bias.reshape(1, d_out))
    return out


# speedup vs baseline: 1.4377x; 1.0334x over previous
import functools

import jax
import jax.numpy as jnp
from jax.experimental import pallas as pl
from jax.experimental.pallas import tpu as pltpu

BM = 2048
BK = 2048


def _fused_body(att_ref, x_ref, w_ref, a_ref, b_ref, o_ref, s_ref,
                *, num_rel, num_k):
    r = pl.program_id(1)
    k = pl.program_id(2)

    @pl.when((r == 0) & (k == 0))
    def _compute_support():
        m = att_ref[0]
        for j in range(1, num_rel):
            m = jnp.maximum(m, att_ref[j])
        denom = jnp.exp(att_ref[0] - m)
        for j in range(1, num_rel):
            denom = denom + jnp.exp(att_ref[j] - m)
        x = x_ref[...]
        for j in range(num_rel):
            att_j = jnp.exp(att_ref[j] - m) / denom
            s_ref[j] = jnp.dot(x, w_ref[j],
                               preferred_element_type=jnp.float32) * att_j
        o_ref[...] = jnp.broadcast_to(b_ref[...], o_ref.shape)

    o_ref[...] += jnp.dot(a_ref[0], s_ref[r, pl.ds(k * BK, BK), :],
                          preferred_element_type=jnp.float32)


def kernel(input, adjs, adj_weight, attention, bias):
    num_rel, n, _ = adjs.shape
    d_in = input.shape[1]
    d_out = adj_weight.shape[2]
    num_k = n // BK

    out = pl.pallas_call(
        functools.partial(_fused_body, num_rel=num_rel, num_k=num_k),
        grid=(n // BM, num_rel, num_k),
        in_specs=[
            pl.BlockSpec(memory_space=pltpu.SMEM),
            pl.BlockSpec((n, d_in), lambda i, r, k: (0, 0)),
            pl.BlockSpec((num_rel, d_in, d_out), lambda i, r, k: (0, 0, 0)),
            pl.BlockSpec((1, BM, BK), lambda i, r, k: (r, i, k)),
            pl.BlockSpec((1, d_out), lambda i, r, k: (0, 0)),
        ],
        out_specs=pl.BlockSpec((BM, d_out), lambda i, r, k: (i, 0)),
        out_shape=jax.ShapeDtypeStruct((n, d_out), jnp.float32),
        scratch_shapes=[pltpu.VMEM((num_rel, n, d_out), jnp.float32)],
        compiler_params=pltpu.CompilerParams(
            dimension_semantics=("parallel", "arbitrary", "arbitrary"),
        ),
    )(attention, input, adj_weight, adjs, bias.reshape(1, d_out))
    return out


# speedup vs baseline: 1.4633x; 1.0178x over previous
import functools

import jax
import jax.numpy as jnp
from jax.experimental import pallas as pl
from jax.experimental.pallas import tpu as pltpu

BM = 4096
BK = 1024


def _fused_body(att_ref, x_ref, w_ref, a_ref, b_ref, o_ref, s_ref,
                *, num_rel, num_k):
    r = pl.program_id(1)
    k = pl.program_id(2)

    @pl.when((r == 0) & (k == 0))
    def _compute_support():
        m = att_ref[0]
        for j in range(1, num_rel):
            m = jnp.maximum(m, att_ref[j])
        denom = jnp.exp(att_ref[0] - m)
        for j in range(1, num_rel):
            denom = denom + jnp.exp(att_ref[j] - m)
        x = x_ref[...]
        for j in range(num_rel):
            att_j = jnp.exp(att_ref[j] - m) / denom
            s_ref[j] = jnp.dot(x, w_ref[j],
                               preferred_element_type=jnp.float32) * att_j
        o_ref[...] = jnp.broadcast_to(b_ref[...], o_ref.shape)

    o_ref[...] += jnp.dot(a_ref[0], s_ref[r, pl.ds(k * BK, BK), :],
                          preferred_element_type=jnp.float32)


def kernel(input, adjs, adj_weight, attention, bias):
    num_rel, n, _ = adjs.shape
    d_in = input.shape[1]
    d_out = adj_weight.shape[2]
    num_k = n // BK

    out = pl.pallas_call(
        functools.partial(_fused_body, num_rel=num_rel, num_k=num_k),
        grid=(n // BM, num_rel, num_k),
        in_specs=[
            pl.BlockSpec(memory_space=pltpu.SMEM),
            pl.BlockSpec((n, d_in), lambda i, r, k: (0, 0)),
            pl.BlockSpec((num_rel, d_in, d_out), lambda i, r, k: (0, 0, 0)),
            pl.BlockSpec((1, BM, BK), lambda i, r, k: (r, i, k)),
            pl.BlockSpec((1, d_out), lambda i, r, k: (0, 0)),
        ],
        out_specs=pl.BlockSpec((BM, d_out), lambda i, r, k: (i, 0)),
        out_shape=jax.ShapeDtypeStruct((n, d_out), jnp.float32),
        scratch_shapes=[pltpu.VMEM((num_rel, n, d_out), jnp.float32)],
        compiler_params=pltpu.CompilerParams(
            dimension_semantics=("parallel", "arbitrary", "arbitrary"),
        ),
    )(attention, input, adj_weight, adjs, bias.reshape(1, d_out))
    return out


# speedup vs baseline: 1.4709x; 1.0052x over previous
import functools

import jax
import jax.numpy as jnp
from jax.experimental import pallas as pl
from jax.experimental.pallas import tpu as pltpu

BM = 4096
BK = 512


def _fused_body(att_ref, x_ref, w_ref, a_ref, b_ref, o_ref, s_ref,
                *, num_rel, num_k):
    r = pl.program_id(1)
    k = pl.program_id(2)

    @pl.when((r == 0) & (k == 0))
    def _compute_support():
        m = att_ref[0]
        for j in range(1, num_rel):
            m = jnp.maximum(m, att_ref[j])
        denom = jnp.exp(att_ref[0] - m)
        for j in range(1, num_rel):
            denom = denom + jnp.exp(att_ref[j] - m)
        x = x_ref[...]
        for j in range(num_rel):
            att_j = jnp.exp(att_ref[j] - m) / denom
            s_ref[j] = jnp.dot(x, w_ref[j],
                               preferred_element_type=jnp.float32) * att_j
        o_ref[...] = jnp.broadcast_to(b_ref[...], o_ref.shape)

    o_ref[...] += jnp.dot(a_ref[0], s_ref[r, pl.ds(k * BK, BK), :],
                          preferred_element_type=jnp.float32)


def kernel(input, adjs, adj_weight, attention, bias):
    num_rel, n, _ = adjs.shape
    d_in = input.shape[1]
    d_out = adj_weight.shape[2]
    num_k = n // BK

    out = pl.pallas_call(
        functools.partial(_fused_body, num_rel=num_rel, num_k=num_k),
        grid=(n // BM, num_rel, num_k),
        in_specs=[
            pl.BlockSpec(memory_space=pltpu.SMEM),
            pl.BlockSpec((n, d_in), lambda i, r, k: (0, 0)),
            pl.BlockSpec((num_rel, d_in, d_out), lambda i, r, k: (0, 0, 0)),
            pl.BlockSpec((1, BM, BK), lambda i, r, k: (r, i, k)),
            pl.BlockSpec((1, d_out), lambda i, r, k: (0, 0)),
        ],
        out_specs=pl.BlockSpec((BM, d_out), lambda i, r, k: (i, 0)),
        out_shape=jax.ShapeDtypeStruct((n, d_out), jnp.float32),
        scratch_shapes=[pltpu.VMEM((num_rel, n, d_out), jnp.float32)],
        compiler_params=pltpu.CompilerParams(
            dimension_semantics=("parallel", "arbitrary", "arbitrary"),
        ),
    )(attention, input, adj_weight, adjs, bias.reshape(1, d_out))
    return out


# speedup vs baseline: 1.4735x; 1.0018x over previous
import functools

import jax
import jax.numpy as jnp
from jax.experimental import pallas as pl
from jax.experimental.pallas import tpu as pltpu

BM = 4096
BK = 512


def _fused_body(att_ref, x_ref, w_ref, a_ref, b_ref, o_ref, s_ref,
                *, num_rel, num_k):
    r = pl.program_id(1)
    k = pl.program_id(2)

    @pl.when((r == 0) & (k == 0))
    def _compute_support():
        m = att_ref[0]
        for j in range(1, num_rel):
            m = jnp.maximum(m, att_ref[j])
        denom = jnp.exp(att_ref[0] - m)
        for j in range(1, num_rel):
            denom = denom + jnp.exp(att_ref[j] - m)
        x = x_ref[...]
        for j in range(num_rel):
            att_j = jnp.exp(att_ref[j] - m) / denom
            s_ref[j] = (jnp.dot(x, w_ref[j], preferred_element_type=jnp.float32)
                        * att_j).astype(jnp.bfloat16)
        o_ref[...] = jnp.broadcast_to(b_ref[...], o_ref.shape)

    o_ref[...] += jnp.dot(a_ref[0].astype(jnp.bfloat16),
                          s_ref[r, pl.ds(k * BK, BK), :],
                          preferred_element_type=jnp.float32)


def kernel(input, adjs, adj_weight, attention, bias):
    num_rel, n, _ = adjs.shape
    d_in = input.shape[1]
    d_out = adj_weight.shape[2]
    num_k = n // BK

    out = pl.pallas_call(
        functools.partial(_fused_body, num_rel=num_rel, num_k=num_k),
        grid=(n // BM, num_rel, num_k),
        in_specs=[
            pl.BlockSpec(memory_space=pltpu.SMEM),
            pl.BlockSpec((n, d_in), lambda i, r, k: (0, 0)),
            pl.BlockSpec((num_rel, d_in, d_out), lambda i, r, k: (0, 0, 0)),
            pl.BlockSpec((1, BM, BK), lambda i, r, k: (r, i, k)),
            pl.BlockSpec((1, d_out), lambda i, r, k: (0, 0)),
        ],
        out_specs=pl.BlockSpec((BM, d_out), lambda i, r, k: (i, 0)),
        out_shape=jax.ShapeDtypeStruct((n, d_out), jnp.float32),
        scratch_shapes=[pltpu.VMEM((num_rel, n, d_out), jnp.bfloat16)],
        compiler_params=pltpu.CompilerParams(
            dimension_semantics=("parallel", "arbitrary", "arbitrary"),
        ),
    )(attention, input, adj_weight, adjs, bias.reshape(1, d_out))
    return out
